# Initial kernel scaffold; baseline (speedup 1.0000x reference)
import jax
import jax.numpy as jnp
from jax.experimental import pallas as pl


def kernel(x, edge_index, batch, init_W1, init_b1, init_g1, init_be1, init_W2, init_b2, init_g2, init_be2, init_lin_W, init_lin_b, gin_W1, gin_b1, gin_g1, gin_be1, gin_W2, gin_b2, gin_g2, gin_be2, eps, lin_W, lin_b):
    raise NotImplementedError("write your pallas kernel here")


# speedup vs baseline: 3.3978x; 3.3978x over previous
import functools

import jax
import jax.numpy as jnp
from jax import lax
from jax.experimental import pallas as pl
from jax.experimental.pallas import tpu as pltpu
from jax.experimental.pallas import tpu_sc as plsc

_G = 64
_BN = 1.0 / (1.0 + 1e-5) ** 0.5


@functools.cache
def _make_sc_agg(n, f, e):
    nw = 32
    epw = e // nw
    chunk = 80
    nchunk = epw // chunk
    assert chunk * nchunk == epw and (chunk % 8) == 0
    rps = (n // (16 * 8)) * 8
    rem = n - 16 * rps
    assert rem % 8 == 0 and rem < rps

    mesh = plsc.VectorSubcoreMesh(core_axis_name="c", subcore_axis_name="s")

    @functools.partial(
        pl.kernel,
        mesh=mesh,
        out_type=jax.ShapeDtypeStruct((2, n, f), jnp.float32),
        scratch_types=[
            pltpu.VMEM((chunk,), jnp.int32),
            pltpu.VMEM((chunk,), jnp.int32),
            pltpu.VMEM((chunk, f), jnp.float32),
            pltpu.VMEM_SHARED((n, f), jnp.float32),
            pltpu.SemaphoreType.DMA,
        ],
    )
    def sc_agg(h_hbm, src_hbm, dst_hbm, out_hbm, srcb, dstb, rows, agg, sem):
        c = lax.axis_index("c")
        s = lax.axis_index("s")
        w = s * 2 + c

        def zbody(i, carry):
            rows[i // 8, pl.ds((i % 8) * 16, 16)] = jnp.zeros((16,), jnp.float32)
            return carry

        lax.fori_loop(0, chunk * (f // 16), zbody, 0)
        r0 = s * rps
        nfull = rps // chunk
        for k in range(nfull):
            pltpu.sync_copy(rows, agg.at[pl.ds(r0 + k * chunk, chunk)])
        tail = rps - nfull * chunk
        if tail:
            pltpu.sync_copy(rows.at[pl.ds(0, tail)],
                            agg.at[pl.ds(r0 + nfull * chunk, tail)])
        if rem:
            @pl.when(s == 15)
            def _zero_rem():
                pltpu.sync_copy(rows.at[pl.ds(0, rem)],
                                agg.at[pl.ds(16 * rps, rem)])
        plsc.subcore_barrier()

        ebase = w * epw

        def cbody(j, carry):
            off = ebase + j * chunk
            pltpu.sync_copy(src_hbm.at[pl.ds(off, chunk)], srcb)
            pltpu.sync_copy(dst_hbm.at[pl.ds(off, chunk)], dstb)
            pltpu.async_copy(h_hbm.at[srcb], rows, sem).wait()
            pltpu.sync_copy(rows, agg.at[dstb], add=True)
            return carry

        lax.fori_loop(0, nchunk, cbody, 0)
        plsc.subcore_barrier()

        pltpu.sync_copy(agg.at[pl.ds(r0, rps)],
                        out_hbm.at[c, pl.ds(r0, rps)])
        if rem:
            @pl.when(s == 15)
            def _write_rem():
                pltpu.sync_copy(agg.at[pl.ds(16 * rps, rem)],
                                out_hbm.at[c, pl.ds(16 * rps, rem)])

    return sc_agg


def _seg_max(vals, batch2d, pooled_ref):
    def body(g, carry):
        m = jnp.max(jnp.where(batch2d == g, vals, float("-inf")), axis=0)
        pooled_ref[pl.ds(g, 1), :] = m[None, :]
        return carry

    lax.fori_loop(0, _G, body, 0)


def _init_body(x_ref, w1_ref, b1_ref, w2_ref, b2_ref, lw_ref, lb_ref,
               batch_ref, h_ref, out0_ref, pooled_ref):
    h1 = jnp.maximum(
        jnp.dot(x_ref[...], w1_ref[...], preferred_element_type=jnp.float32)
        + b1_ref[...], 0.0)
    h = jnp.maximum(
        jnp.dot(h1, w2_ref[...], preferred_element_type=jnp.float32)
        + b2_ref[...], 0.0)
    h_ref[...] = h
    y = (jnp.dot(h, lw_ref[...], preferred_element_type=jnp.float32)
         + lb_ref[...])
    _seg_max(y, batch_ref[...], pooled_ref)
    out0_ref[...] = pooled_ref[...]


def _layer_body(h_ref, p_ref, eps_ref, w1_ref, b1_ref, w2_ref, b2_ref,
                lw_ref, lb_ref, batch_ref, out_ref,
                h_out_ref, out_out_ref, pooled_ref):
    z = (1.0 + eps_ref[0, 0]) * h_ref[...] + p_ref[0] + p_ref[1]
    z1 = jnp.maximum(
        jnp.dot(z, w1_ref[...], preferred_element_type=jnp.float32)
        + b1_ref[...], 0.0)
    hn = jnp.maximum(
        jnp.dot(z1, w2_ref[...], preferred_element_type=jnp.float32)
        + b2_ref[...], 0.0)
    h_out_ref[...] = hn
    _seg_max(hn, batch_ref[...], pooled_ref)
    out_out_ref[...] = (
        out_ref[...]
        + jnp.dot(pooled_ref[...], lw_ref[...],
                  preferred_element_type=jnp.float32)
        + lb_ref[...])


def _tc_init(x, w1, b1, w2, b2, lw, lb, batch2d):
    n, _ = x.shape
    emb = w1.shape[1]
    ncls = lw.shape[1]
    return pl.pallas_call(
        _init_body,
        out_shape=[
            jax.ShapeDtypeStruct((n, emb), jnp.float32),
            jax.ShapeDtypeStruct((_G, ncls), jnp.float32),
        ],
        scratch_shapes=[pltpu.VMEM((_G, ncls), jnp.float32)],
    )(x, w1, b1, w2, b2, lw, lb, batch2d)


def _tc_layer(h, parts, eps_i, w1, b1, w2, b2, lw, lb, batch2d, out):
    n, emb = h.shape
    ncls = lw.shape[1]
    return pl.pallas_call(
        _layer_body,
        out_shape=[
            jax.ShapeDtypeStruct((n, emb), jnp.float32),
            jax.ShapeDtypeStruct((_G, ncls), jnp.float32),
        ],
        scratch_shapes=[pltpu.VMEM((_G, emb), jnp.float32)],
    )(h, parts, eps_i, w1, b1, w2, b2, lw, lb, batch2d, out)


def kernel(x, edge_index, batch, init_W1, init_b1, init_g1, init_be1,
           init_W2, init_b2, init_g2, init_be2, init_lin_W, init_lin_b,
           gin_W1, gin_b1, gin_g1, gin_be1, gin_W2, gin_b2, gin_g2, gin_be2,
           eps, lin_W, lin_b):
    n, _ = x.shape
    e = edge_index.shape[1]
    emb = init_W1.shape[1]
    nlayers = gin_W1.shape[0]

    a1 = init_g1 * _BN
    w1 = init_W1 * a1[None, :]
    b1 = (init_b1 * a1 + init_be1)[None, :]
    a2 = init_g2 * _BN
    w2 = init_W2 * a2[None, :]
    b2 = (init_b2 * a2 + init_be2)[None, :]
    ga1 = gin_g1 * _BN
    gw1 = gin_W1 * ga1[:, None, :]
    gb1 = gin_b1 * ga1 + gin_be1
    ga2 = gin_g2 * _BN
    gw2 = gin_W2 * ga2[:, None, :]
    gb2 = gin_b2 * ga2 + gin_be2

    src = edge_index[0]
    dst = edge_index[1]
    batch2d = batch[:, None]

    h, out = _tc_init(x, w1, b1, w2, b2, init_lin_W, init_lin_b[None, :],
                      batch2d)
    sc_agg = _make_sc_agg(n, emb, e)
    for i in range(nlayers):
        parts = sc_agg(h, src, dst)
        h, out = _tc_layer(h, parts, eps[i].reshape(1, 1),
                           gw1[i], gb1[i][None, :], gw2[i], gb2[i][None, :],
                           lin_W[i], lin_b[i][None, :], batch2d, out)
    return out


# speedup vs baseline: 5.1477x; 1.5150x over previous
import functools

import jax
import jax.numpy as jnp
from jax import lax
from jax.experimental import pallas as pl
from jax.experimental.pallas import tpu as pltpu
from jax.experimental.pallas import tpu_sc as plsc

_G = 64
_BN = 1.0 / (1.0 + 1e-5) ** 0.5


@functools.cache
def _make_sc_agg(n, f, e):
    nw = 32
    epw = e // nw
    chunk = 80
    nchunk = epw // chunk
    assert chunk * nchunk == epw and (chunk % 8) == 0
    rps = (n // (16 * 8)) * 8
    rem = n - 16 * rps
    assert rem % 8 == 0 and rem < rps

    mesh = plsc.VectorSubcoreMesh(core_axis_name="c", subcore_axis_name="s")

    assert nchunk % 2 == 1 and nchunk >= 5

    @functools.partial(
        pl.kernel,
        mesh=mesh,
        out_type=jax.ShapeDtypeStruct((2, n, f), jnp.float32),
        scratch_types=[
            pltpu.VMEM((chunk,), jnp.int32),
            pltpu.VMEM((chunk,), jnp.int32),
            pltpu.VMEM((chunk, f), jnp.float32),
            pltpu.VMEM((chunk,), jnp.int32),
            pltpu.VMEM((chunk,), jnp.int32),
            pltpu.VMEM((chunk, f), jnp.float32),
            pltpu.VMEM_SHARED((n, f), jnp.float32),
            pltpu.SemaphoreType.DMA,
            pltpu.SemaphoreType.DMA,
            pltpu.SemaphoreType.DMA,
            pltpu.SemaphoreType.DMA,
            pltpu.SemaphoreType.DMA,
            pltpu.SemaphoreType.DMA,
        ],
    )
    def sc_agg(h_hbm, src_hbm, dst_hbm, out_hbm,
               srcA, dstA, rowsA, srcB, dstB, rowsB, agg,
               semAs, semAd, semAg, semBs, semBd, semBg):
        c = lax.axis_index("c")
        s = lax.axis_index("s")
        w = s * 2 + c
        rows = rowsA

        def zbody(i, carry):
            rows[i // 8, pl.ds((i % 8) * 16, 16)] = jnp.zeros((16,), jnp.float32)
            return carry

        lax.fori_loop(0, chunk * (f // 16), zbody, 0)
        r0 = s * rps
        nfull = rps // chunk
        for k in range(nfull):
            pltpu.sync_copy(rows, agg.at[pl.ds(r0 + k * chunk, chunk)])
        tail = rps - nfull * chunk
        if tail:
            pltpu.sync_copy(rows.at[pl.ds(0, tail)],
                            agg.at[pl.ds(r0 + nfull * chunk, tail)])
        if rem:
            @pl.when(s == 15)
            def _zero_rem():
                pltpu.sync_copy(rows.at[pl.ds(0, rem)],
                                agg.at[pl.ds(16 * rps, rem)])
        plsc.subcore_barrier()

        ebase = w * epw
        slotA = (srcA, dstA, rowsA, semAs, semAd, semAg)
        slotB = (srcB, dstB, rowsB, semBs, semBd, semBg)

        def idx_issue(wi, slot):
            srcb, dstb, _, sems, semd, _ = slot
            off = ebase + wi * chunk
            pltpu.async_copy(src_hbm.at[pl.ds(off, chunk)], srcb, sems)
            pltpu.async_copy(dst_hbm.at[pl.ds(off, chunk)], dstb, semd)

        def idx_wait(wi, slot):
            srcb, dstb, _, sems, semd, _ = slot
            off = ebase + wi * chunk
            pltpu.make_async_copy(src_hbm.at[pl.ds(off, chunk)], srcb, sems).wait()
            pltpu.make_async_copy(dst_hbm.at[pl.ds(off, chunk)], dstb, semd).wait()

        def gather_issue(slot):
            srcb, _, rws, _, _, semg = slot
            pltpu.async_copy(h_hbm.at[srcb], rws, semg)

        def gather_wait(slot):
            srcb, _, rws, _, _, semg = slot
            pltpu.make_async_copy(h_hbm.at[srcb], rws, semg).wait()

        def scatter(slot):
            _, dstb, rws, _, _, _ = slot
            pltpu.sync_copy(rws, agg.at[dstb], add=True)

        idx_issue(0, slotA)
        idx_issue(1, slotB)
        idx_wait(0, slotA)
        gather_issue(slotA)

        def process(wi, slot_p, slot_q, nxt_gather, nxt_idx):
            gather_wait(slot_p)
            if nxt_gather:
                idx_wait(wi + 1, slot_q)
                gather_issue(slot_q)
            scatter(slot_p)
            if nxt_idx:
                idx_issue(wi + 2, slot_p)

        def cbody(t, carry):
            w0 = 2 * t
            process(w0, slotA, slotB, True, True)
            process(w0 + 1, slotB, slotA, True, True)
            return carry

        lax.fori_loop(0, (nchunk - 3) // 2, cbody, 0)
        process(nchunk - 3, slotA, slotB, True, True)
        process(nchunk - 2, slotB, slotA, True, False)
        process(nchunk - 1, slotA, slotB, False, False)
        plsc.subcore_barrier()

        pltpu.sync_copy(agg.at[pl.ds(r0, rps)],
                        out_hbm.at[c, pl.ds(r0, rps)])
        if rem:
            @pl.when(s == 15)
            def _write_rem():
                pltpu.sync_copy(agg.at[pl.ds(16 * rps, rem)],
                                out_hbm.at[c, pl.ds(16 * rps, rem)])

    return sc_agg


def _seg_max(vals, batch2d, pooled_ref):
    def body(g, carry):
        m = jnp.max(jnp.where(batch2d == g, vals, float("-inf")), axis=0)
        pooled_ref[pl.ds(g, 1), :] = m[None, :]
        return carry

    lax.fori_loop(0, _G, body, 0)


def _init_body(x_ref, w1_ref, b1_ref, w2_ref, b2_ref, lw_ref, lb_ref,
               batch_ref, h_ref, out0_ref, pooled_ref):
    h1 = jnp.maximum(
        jnp.dot(x_ref[...], w1_ref[...], preferred_element_type=jnp.float32)
        + b1_ref[...], 0.0)
    h = jnp.maximum(
        jnp.dot(h1, w2_ref[...], preferred_element_type=jnp.float32)
        + b2_ref[...], 0.0)
    h_ref[...] = h
    y = (jnp.dot(h, lw_ref[...], preferred_element_type=jnp.float32)
         + lb_ref[...])
    _seg_max(y, batch_ref[...], pooled_ref)
    out0_ref[...] = pooled_ref[...]


def _layer_body(h_ref, p_ref, eps_ref, w1_ref, b1_ref, w2_ref, b2_ref,
                lw_ref, lb_ref, batch_ref, out_ref,
                h_out_ref, out_out_ref, pooled_ref):
    z = (1.0 + eps_ref[0, 0]) * h_ref[...] + p_ref[0] + p_ref[1]
    z1 = jnp.maximum(
        jnp.dot(z, w1_ref[...], preferred_element_type=jnp.float32)
        + b1_ref[...], 0.0)
    hn = jnp.maximum(
        jnp.dot(z1, w2_ref[...], preferred_element_type=jnp.float32)
        + b2_ref[...], 0.0)
    h_out_ref[...] = hn
    _seg_max(hn, batch_ref[...], pooled_ref)
    out_out_ref[...] = (
        out_ref[...]
        + jnp.dot(pooled_ref[...], lw_ref[...],
                  preferred_element_type=jnp.float32)
        + lb_ref[...])


def _tc_init(x, w1, b1, w2, b2, lw, lb, batch2d):
    n, _ = x.shape
    emb = w1.shape[1]
    ncls = lw.shape[1]
    return pl.pallas_call(
        _init_body,
        out_shape=[
            jax.ShapeDtypeStruct((n, emb), jnp.float32),
            jax.ShapeDtypeStruct((_G, ncls), jnp.float32),
        ],
        scratch_shapes=[pltpu.VMEM((_G, ncls), jnp.float32)],
    )(x, w1, b1, w2, b2, lw, lb, batch2d)


def _tc_layer(h, parts, eps_i, w1, b1, w2, b2, lw, lb, batch2d, out):
    n, emb = h.shape
    ncls = lw.shape[1]
    return pl.pallas_call(
        _layer_body,
        out_shape=[
            jax.ShapeDtypeStruct((n, emb), jnp.float32),
            jax.ShapeDtypeStruct((_G, ncls), jnp.float32),
        ],
        scratch_shapes=[pltpu.VMEM((_G, emb), jnp.float32)],
    )(h, parts, eps_i, w1, b1, w2, b2, lw, lb, batch2d, out)


def kernel(x, edge_index, batch, init_W1, init_b1, init_g1, init_be1,
           init_W2, init_b2, init_g2, init_be2, init_lin_W, init_lin_b,
           gin_W1, gin_b1, gin_g1, gin_be1, gin_W2, gin_b2, gin_g2, gin_be2,
           eps, lin_W, lin_b):
    n, _ = x.shape
    e = edge_index.shape[1]
    emb = init_W1.shape[1]
    nlayers = gin_W1.shape[0]

    a1 = init_g1 * _BN
    w1 = init_W1 * a1[None, :]
    b1 = (init_b1 * a1 + init_be1)[None, :]
    a2 = init_g2 * _BN
    w2 = init_W2 * a2[None, :]
    b2 = (init_b2 * a2 + init_be2)[None, :]
    ga1 = gin_g1 * _BN
    gw1 = gin_W1 * ga1[:, None, :]
    gb1 = gin_b1 * ga1 + gin_be1
    ga2 = gin_g2 * _BN
    gw2 = gin_W2 * ga2[:, None, :]
    gb2 = gin_b2 * ga2 + gin_be2

    src = edge_index[0]
    dst = edge_index[1]
    batch2d = batch[:, None]

    h, out = _tc_init(x, w1, b1, w2, b2, init_lin_W, init_lin_b[None, :],
                      batch2d)
    sc_agg = _make_sc_agg(n, emb, e)
    for i in range(nlayers):
        parts = sc_agg(h, src, dst)
        h, out = _tc_layer(h, parts, eps[i].reshape(1, 1),
                           gw1[i], gb1[i][None, :], gw2[i], gb2[i][None, :],
                           lin_W[i], lin_b[i][None, :], batch2d, out)
    return out
